# baseline (speedup 1.0000x reference)
import jax
import jax.numpy as jnp
import numpy as np
from jax import lax
from jax.experimental import pallas as pl
from jax.experimental.pallas import tpu as pltpu
from jax.experimental.pallas import tpu_sc as plsc

N = 10000
E = 320000
D = 128
OUT = 10

NC = 2
NS = 16
NW = NC * NS
CH = 32
IDXR = 79
EW = IDXR * 128
E_PAD = NW * EW
N_PAD = 10240
ROWS_PER_TILE = N_PAD // NS
ZR = 8
DEN_R = N_PAD // D

_SCALE = np.float32(1.0 / np.sqrt(np.float32(D)))
_LN_C = np.float32(1.0 / np.sqrt(1.0 + 1e-05))


def _edge_kernel_body(q_hbm, k_hbm, v_hbm, src_hbm, dst_hbm,
                      agg_hbm, den_hbm, *refs):
    (qr0, qr1, kr0, kr1, vr0, vr1, ob0, ob1, oh0, oh1,
     sc0, sc1, sc2, sc3, dc0, dc1, dc2, dc3, dx0, dx1, dx2, dx3,
     dh0, dh1, dh2, dh3, rbs_v, rbd_v,
     red_f, red_i, zbuf, acc, den,
     sem_g0, sem_g1, sem_s0, sem_s1, sem_r) = refs
    qr = (qr0, qr1)
    kr = (kr0, kr1)
    vr = (vr0, vr1)
    ob = (ob0, ob1)
    oh = (oh0, oh1)
    scs = (sc0, sc1, sc2, sc3)
    dcs = (dc0, dc1, dc2, dc3)
    dxs = (dx0, dx1, dx2, dx3)
    dhs = (dh0, dh1, dh2, dh3)
    sem_g = (sem_g0, sem_g1)
    sem_s = (sem_s0, sem_s1)

    cid = lax.axis_index("c")
    sid = lax.axis_index("s")
    wid = sid * NC + cid

    zero16 = jnp.zeros((16,), jnp.float32)
    iota16 = lax.iota(jnp.int32, 16)

    def _zfill(i, _):
        for j in range(D // 16):
            zbuf[i, pl.ds(16 * j, 16)] = zero16
        return 0
    lax.fori_loop(0, ZR, _zfill, 0, unroll=4)

    for t in range(ROWS_PER_TILE // ZR):
        pltpu.sync_copy(zbuf, acc.at[pl.ds(sid * ROWS_PER_TILE + t * ZR, ZR)])

    @pl.when(sid < DEN_R // 8)
    def _():
        pltpu.sync_copy(zbuf, den.at[pl.ds(sid * 8, 8)])
    plsc.subcore_barrier()

    def _prep(row, k4p):
        rsel = row % 2
        base = k4p * CH
        for g in range(CH // 16):
            dv = rbd_v[rsel, pl.ds(base + 16 * g, 16)]
            sv = rbs_v[rsel, pl.ds(base + 16 * g, 16)]
            scs[k4p][pl.ds(16 * g, 16)] = sv
            dcs[k4p][pl.ds(16 * g, 16)] = dv
            dxs[k4p][pl.ds(16 * g, 16)] = dv
            dhs[k4p][pl.ds(16 * g, 16)] = lax.shift_right_logical(dv, 7)

    def _issue_gathers(k4p, s):
        cq = pltpu.async_copy(q_hbm.at[dcs[k4p]], qr[s], sem_g[s])
        ck = pltpu.async_copy(k_hbm.at[scs[k4p]], kr[s], sem_g[s])
        cv = pltpu.async_copy(v_hbm.at[scs[k4p]], vr[s], sem_g[s])
        return cq, ck, cv

    def _wait_gathers(k4p, s):
        pltpu.make_async_copy(q_hbm.at[dcs[k4p]], qr[s], sem_g[s]).wait()
        pltpu.make_async_copy(k_hbm.at[scs[k4p]], kr[s], sem_g[s]).wait()
        pltpu.make_async_copy(v_hbm.at[scs[k4p]], vr[s], sem_g[s]).wait()

    def _issue_scatters(k4p, s):
        pltpu.async_copy(ob[s], acc.at[dcs[k4p]], sem_s[s], add=True)
        pltpu.async_copy(oh[s], den.at[dhs[k4p]], sem_s[s], add=True)

    def _wait_scatters(k4p, s):
        pltpu.make_async_copy(ob[s], acc.at[dcs[k4p]], sem_s[s]).wait()
        pltpu.make_async_copy(oh[s], den.at[dhs[k4p]], sem_s[s]).wait()

    def _compute(k4p, s):
        def _edge(e, _):
            acc_d = qr[s][e, pl.ds(0, 16)] * kr[s][e, pl.ds(0, 16)]
            for j in range(1, D // 16):
                acc_d = acc_d + (qr[s][e, pl.ds(16 * j, 16)]
                                 * kr[s][e, pl.ds(16 * j, 16)])
            rb = e * 32
            for sh in (8, 4, 2, 1):
                red_f[pl.ds(rb, 16)] = acc_d
                red_f[pl.ds(rb + 16, 16)] = acc_d
                acc_d = acc_d + red_f[pl.ds(rb + sh, 16)]
            ex = jnp.exp(acc_d * _SCALE)
            dsplat = dxs[k4p][pl.ds(e, 16)]
            for sh in (1, 2, 4, 8):
                red_i[pl.ds(rb + sh, 16)] = dsplat
                w = red_i[pl.ds(rb, 16)]
                dsplat = jnp.where((iota16 >= sh) & (iota16 < 2 * sh),
                                   w, dsplat)
            rlow = dsplat & (D - 1)
            for j in range(D // 16):
                ob[s][e, pl.ds(16 * j, 16)] = ex * vr[s][e, pl.ds(16 * j, 16)]
                oh[s][e, pl.ds(16 * j, 16)] = jnp.where(
                    iota16 + 16 * j == rlow, ex, zero16)
            return 0

        lax.fori_loop(0, CH, _edge, 0, unroll=2)

    pltpu.sync_copy(src_hbm.at[wid, 0], rbs_v.at[0])
    pltpu.sync_copy(dst_hbm.at[wid, 0], rbd_v.at[0])
    pltpu.async_copy(src_hbm.at[wid, 1], rbs_v.at[1], sem_r)
    pltpu.async_copy(dst_hbm.at[wid, 1], rbd_v.at[1], sem_r)
    _prep(0, 0)
    _issue_gathers(0, 0)

    def _row(row, _):
        for k4 in range(4):
            s = k4 % 2
            if k4 == 3:
                @pl.when(row < IDXR - 1)
                def _():
                    pltpu.make_async_copy(src_hbm.at[wid, 0],
                                          rbs_v.at[0], sem_r).wait()
                    pltpu.make_async_copy(dst_hbm.at[wid, 0],
                                          rbd_v.at[0], sem_r).wait()
                    _prep(row + 1, 0)
                    _issue_gathers(0, 1 - s)

                @pl.when(row < IDXR - 2)
                def _():
                    pltpu.async_copy(src_hbm.at[wid, row + 2],
                                     rbs_v.at[row % 2], sem_r)
                    pltpu.async_copy(dst_hbm.at[wid, row + 2],
                                     rbd_v.at[row % 2], sem_r)
            else:
                _prep(row, k4 + 1)
                _issue_gathers(k4 + 1, 1 - s)
            _wait_gathers(k4, s)
            if k4 >= 2:
                _wait_scatters(k4 - 2, s)
            else:
                @pl.when(row > 0)
                def _():
                    _wait_scatters(k4 + 2, s)
            _compute(k4, s)
            _issue_scatters(k4, s)
        return 0

    lax.fori_loop(0, IDXR, _row, 0)
    _wait_scatters(2, 0)
    _wait_scatters(3, 1)

    plsc.subcore_barrier()
    pltpu.sync_copy(
        acc.at[pl.ds(sid * ROWS_PER_TILE, ROWS_PER_TILE)],
        agg_hbm.at[cid, pl.ds(sid * ROWS_PER_TILE, ROWS_PER_TILE)])

    @pl.when(sid < DEN_R // 8)
    def _():
        pltpu.sync_copy(den.at[pl.ds(sid * 8, 8)],
                        den_hbm.at[cid, pl.ds(sid * 8, 8)])


def _edge_partials(q, k, v, src, dst):
    mesh = plsc.VectorSubcoreMesh(core_axis_name="c", subcore_axis_name="s",
                                  num_cores=NC, num_subcores=NS)
    f32 = jnp.float32
    i32 = jnp.int32
    fn = pl.kernel(
        _edge_kernel_body,
        out_type=(jax.ShapeDtypeStruct((NC, N_PAD, D), f32),
                  jax.ShapeDtypeStruct((NC, DEN_R, D), f32)),
        mesh=mesh,
        scratch_types=(
            [pltpu.VMEM((CH, D), f32)] * 10
            + [pltpu.VMEM((CH,), i32)] * 8
            + [pltpu.VMEM((CH + 16,), i32)] * 4
            + [pltpu.VMEM((CH,), i32)] * 4
            + [pltpu.VMEM((2, 128), i32)] * 2
            + [pltpu.VMEM((CH * 32,), f32),
               pltpu.VMEM((CH * 32,), i32),
               pltpu.VMEM((ZR, D), f32),
               pltpu.VMEM_SHARED((N_PAD, D), f32),
               pltpu.VMEM_SHARED((DEN_R, D), f32)]
            + [pltpu.SemaphoreType.DMA] * 5
        ),
    )
    return fn(q, k, v, src, dst)


_BR = 1024


def _qkv_body(x_ref, wq_ref, bq_ref, wk_ref, bk_ref, wv_ref, bv_ref,
              q_ref, k_ref, v_ref):
    xb = x_ref[...]
    q_ref[...] = jnp.dot(xb, wq_ref[...], preferred_element_type=jnp.float32) + bq_ref[...]
    k_ref[...] = jnp.dot(xb, wk_ref[...], preferred_element_type=jnp.float32) + bk_ref[...]
    v_ref[...] = jnp.dot(xb, wv_ref[...], preferred_element_type=jnp.float32) + bv_ref[...]


def _qkv(x, Wqt, bq, Wkt, bk, Wvt, bv):
    grid = (N_PAD // _BR,)
    row_spec = pl.BlockSpec((_BR, D), lambda i: (i, 0))
    w_spec = pl.BlockSpec((D, D), lambda i: (0, 0))
    b_spec = pl.BlockSpec((1, D), lambda i: (0, 0))
    return pl.pallas_call(
        _qkv_body,
        grid=grid,
        in_specs=[row_spec, w_spec, b_spec, w_spec, b_spec, w_spec, b_spec],
        out_specs=[row_spec, row_spec, row_spec],
        out_shape=[jax.ShapeDtypeStruct((N_PAD, D), jnp.float32)] * 3,
    )(x, Wqt, bq.reshape(1, D), Wkt, bk.reshape(1, D), Wvt, bv.reshape(1, D))


def _epi_body(agg_ref, den_ref, x_ref, ws_ref, bs_ref, h_ref):
    agg = agg_ref[0] + agg_ref[1]
    denom = den_ref[0] + den_ref[1] + np.float32(1e-16)
    h = agg / denom + jnp.dot(x_ref[...], ws_ref[...],
                              preferred_element_type=jnp.float32) + bs_ref[...]
    h_ref[...] = jnp.maximum(h, 0.0)


def _epilogue(agg, den_col, x, Wst, bs):
    grid = (N_PAD // _BR,)
    return pl.pallas_call(
        _epi_body,
        grid=grid,
        in_specs=[
            pl.BlockSpec((NC, _BR, D), lambda i: (0, i, 0)),
            pl.BlockSpec((NC, _BR, 1), lambda i: (0, i, 0)),
            pl.BlockSpec((_BR, D), lambda i: (i, 0)),
            pl.BlockSpec((D, D), lambda i: (0, 0)),
            pl.BlockSpec((1, D), lambda i: (0, 0)),
        ],
        out_specs=pl.BlockSpec((_BR, D), lambda i: (i, 0)),
        out_shape=jax.ShapeDtypeStruct((N_PAD, D), jnp.float32),
    )(agg, den_col, x, Wst, bs.reshape(1, D))


def _head_body(h_ref, w1_ref, b1_ref, w2_ref, b2_ref, o_ref):
    g = jnp.mean(h_ref[...], axis=0, keepdims=True)
    z = jnp.dot(g, w1_ref[...], preferred_element_type=jnp.float32) + b1_ref[...]
    z = jnp.maximum(z * _LN_C, 0.0)
    o_ref[...] = jnp.dot(z, w2_ref[...], preferred_element_type=jnp.float32) + b2_ref[...]


def _head(h, Wc1t, bc1, Wc2t, bc2):
    return pl.pallas_call(
        _head_body,
        out_shape=jax.ShapeDtypeStruct((1, OUT), jnp.float32),
    )(h, Wc1t, bc1.reshape(1, D // 2), Wc2t, bc2.reshape(1, OUT))


def kernel(x, edge_index, Wq0, bq0, Wk0, bk0, Wv0, bv0, Ws0, bs0,
           Wq1, bq1, Wk1, bk1, Wv1, bv1, Ws1, bs1, Wc1, bc1, Wc2, bc2):
    pad = jnp.full((E_PAD - E,), N, jnp.int32)
    src = jnp.concatenate([edge_index[0], pad]).reshape(NW, IDXR, 128)
    dst = jnp.concatenate([edge_index[1], pad]).reshape(NW, IDXR, 128)
    xp = jnp.pad(x, ((0, N_PAD - N), (0, 0)))

    def layer(h, Wq, bq, Wk, bk, Wv, bv, Ws, bs):
        q, k, v = _qkv(h, Wq.T, bq, Wk.T, bk, Wv.T, bv)
        agg, den = _edge_partials(q, k, v, src, dst)
        den_col = den.reshape(NC, N_PAD, 1)
        return _epilogue(agg, den_col, h, Ws.T, bs)

    h1 = layer(xp, Wq0, bq0, Wk0, bk0, Wv0, bv0, Ws0, bs0)
    h2 = layer(h1, Wq1, bq1, Wk1, bk1, Wv1, bv1, Ws1, bs1)
    return _head(h2[:N], Wc1.T, bc1, Wc2.T, bc2)


# speedup vs baseline: 1.7182x; 1.7182x over previous
import jax
import jax.numpy as jnp
import numpy as np
from jax import lax
from jax.experimental import pallas as pl
from jax.experimental.pallas import tpu as pltpu
from jax.experimental.pallas import tpu_sc as plsc

N = 10000
E = 320000
D = 128
OUT = 10

NC = 2
NS = 16
NW = NC * NS
CH = 32
IDXR = 79
EW = IDXR * 128
E_PAD = NW * EW
N_PAD = 10240
ROWS_PER_TILE = N_PAD // NS
ZR = 8
DEN_R = N_PAD // D

_SCALE = np.float32(1.0 / np.sqrt(np.float32(D)))
_LN_C = np.float32(1.0 / np.sqrt(1.0 + 1e-05))


def _edge_kernel_body(q_hbm, k_hbm, v_hbm, src_hbm, dst_hbm,
                      agg_hbm, den_hbm, *refs):
    (qr0, qr1, kr0, kr1, vr0, vr1, ob0, ob1, oh0, oh1,
     sc0, sc1, sc2, sc3, dc0, dc1, dc2, dc3, dx0, dx1, dx2, dx3,
     dh0, dh1, dh2, dh3, rbs_v, rbd_v,
     red_f, red_i, zbuf, acc, den,
     sem_g0, sem_g1, sem_s0, sem_s1, sem_r) = refs
    qr = (qr0, qr1)
    kr = (kr0, kr1)
    vr = (vr0, vr1)
    ob = (ob0, ob1)
    oh = (oh0, oh1)
    scs = (sc0, sc1, sc2, sc3)
    dcs = (dc0, dc1, dc2, dc3)
    dxs = (dx0, dx1, dx2, dx3)
    dhs = (dh0, dh1, dh2, dh3)
    sem_g = (sem_g0, sem_g1)
    sem_s = (sem_s0, sem_s1)

    cid = lax.axis_index("c")
    sid = lax.axis_index("s")
    wid = sid * NC + cid

    zero16 = jnp.zeros((16,), jnp.float32)
    iota16 = lax.iota(jnp.int32, 16)

    def _zfill(i, _):
        for j in range(D // 16):
            zbuf[i, pl.ds(16 * j, 16)] = zero16
        return 0
    lax.fori_loop(0, ZR, _zfill, 0, unroll=4)

    for t in range(ROWS_PER_TILE // ZR):
        pltpu.sync_copy(zbuf, acc.at[pl.ds(sid * ROWS_PER_TILE + t * ZR, ZR)])

    @pl.when(sid < DEN_R // 8)
    def _():
        pltpu.sync_copy(zbuf, den.at[pl.ds(sid * 8, 8)])
    plsc.subcore_barrier()

    def _prep(row, k4p):
        rsel = row % 2
        base = k4p * CH
        for g in range(CH // 16):
            dv = rbd_v[rsel, pl.ds(base + 16 * g, 16)]
            sv = rbs_v[rsel, pl.ds(base + 16 * g, 16)]
            scs[k4p][pl.ds(16 * g, 16)] = sv
            dcs[k4p][pl.ds(16 * g, 16)] = dv
            dxs[k4p][pl.ds(16 * g, 16)] = dv
            dhs[k4p][pl.ds(16 * g, 16)] = lax.shift_right_logical(dv, 7)

    def _issue_gathers(k4p, s):
        cq = pltpu.async_copy(q_hbm.at[dcs[k4p]], qr[s], sem_g[s])
        ck = pltpu.async_copy(k_hbm.at[scs[k4p]], kr[s], sem_g[s])
        cv = pltpu.async_copy(v_hbm.at[scs[k4p]], vr[s], sem_g[s])
        return cq, ck, cv

    def _wait_gathers(k4p, s):
        pltpu.make_async_copy(q_hbm.at[dcs[k4p]], qr[s], sem_g[s]).wait()
        pltpu.make_async_copy(k_hbm.at[scs[k4p]], kr[s], sem_g[s]).wait()
        pltpu.make_async_copy(v_hbm.at[scs[k4p]], vr[s], sem_g[s]).wait()

    def _issue_scatters(k4p, s):
        pltpu.async_copy(ob[s], acc.at[dcs[k4p]], sem_s[s], add=True)
        pltpu.async_copy(oh[s], den.at[dhs[k4p]], sem_s[s], add=True)

    def _wait_scatters(k4p, s):
        pltpu.make_async_copy(ob[s], acc.at[dcs[k4p]], sem_s[s]).wait()
        pltpu.make_async_copy(oh[s], den.at[dhs[k4p]], sem_s[s]).wait()

    def _compute(k4p, s):
        U = 8

        def _edge4(eg, _):
            accs = []
            for u in range(U):
                e = eg * U + u
                a = qr[s][e, pl.ds(0, 16)] * kr[s][e, pl.ds(0, 16)]
                for j in range(1, D // 16):
                    a = a + (qr[s][e, pl.ds(16 * j, 16)]
                             * kr[s][e, pl.ds(16 * j, 16)])
                accs.append(a)
            for sh in (8, 4, 2, 1):
                for u in range(U):
                    red_f[pl.ds(u * 32, 16)] = accs[u]
                    red_f[pl.ds(u * 32 + 16, 16)] = accs[u]
                for u in range(U):
                    accs[u] = accs[u] + red_f[pl.ds(u * 32 + sh, 16)]
            exs = [jnp.exp(a * _SCALE) for a in accs]
            dsp = [dxs[k4p][pl.ds(eg * U + u, 16)] for u in range(U)]
            for sh in (1, 2, 4, 8):
                for u in range(U):
                    red_i[pl.ds(u * 32 + sh, 16)] = dsp[u]
                for u in range(U):
                    dsp[u] = jnp.where((iota16 >= sh) & (iota16 < 2 * sh),
                                       red_i[pl.ds(u * 32, 16)], dsp[u])
            for u in range(U):
                e = eg * U + u
                rlow = dsp[u] & (D - 1)
                ex = exs[u]
                for j in range(D // 16):
                    ob[s][e, pl.ds(16 * j, 16)] = ex * vr[s][e, pl.ds(16 * j, 16)]
                    oh[s][e, pl.ds(16 * j, 16)] = jnp.where(
                        iota16 + 16 * j == rlow, ex, zero16)
            return 0

        lax.fori_loop(0, CH // U, _edge4, 0)

    pltpu.sync_copy(src_hbm.at[wid, 0], rbs_v.at[0])
    pltpu.sync_copy(dst_hbm.at[wid, 0], rbd_v.at[0])
    pltpu.async_copy(src_hbm.at[wid, 1], rbs_v.at[1], sem_r)
    pltpu.async_copy(dst_hbm.at[wid, 1], rbd_v.at[1], sem_r)
    _prep(0, 0)
    _issue_gathers(0, 0)

    def _row(row, _):
        for k4 in range(4):
            s = k4 % 2
            if k4 == 3:
                @pl.when(row < IDXR - 1)
                def _():
                    pltpu.make_async_copy(src_hbm.at[wid, 0],
                                          rbs_v.at[0], sem_r).wait()
                    pltpu.make_async_copy(dst_hbm.at[wid, 0],
                                          rbd_v.at[0], sem_r).wait()
                    _prep(row + 1, 0)
                    _issue_gathers(0, 1 - s)

                @pl.when(row < IDXR - 2)
                def _():
                    pltpu.async_copy(src_hbm.at[wid, row + 2],
                                     rbs_v.at[row % 2], sem_r)
                    pltpu.async_copy(dst_hbm.at[wid, row + 2],
                                     rbd_v.at[row % 2], sem_r)
            else:
                _prep(row, k4 + 1)
                _issue_gathers(k4 + 1, 1 - s)
            _wait_gathers(k4, s)
            if k4 >= 2:
                _wait_scatters(k4 - 2, s)
            else:
                @pl.when(row > 0)
                def _():
                    _wait_scatters(k4 + 2, s)
            _compute(k4, s)
            _issue_scatters(k4, s)
        return 0

    lax.fori_loop(0, IDXR, _row, 0)
    _wait_scatters(2, 0)
    _wait_scatters(3, 1)

    plsc.subcore_barrier()
    pltpu.sync_copy(
        acc.at[pl.ds(sid * ROWS_PER_TILE, ROWS_PER_TILE)],
        agg_hbm.at[cid, pl.ds(sid * ROWS_PER_TILE, ROWS_PER_TILE)])

    @pl.when(sid < DEN_R // 8)
    def _():
        pltpu.sync_copy(den.at[pl.ds(sid * 8, 8)],
                        den_hbm.at[cid, pl.ds(sid * 8, 8)])


def _edge_partials(q, k, v, src, dst):
    mesh = plsc.VectorSubcoreMesh(core_axis_name="c", subcore_axis_name="s",
                                  num_cores=NC, num_subcores=NS)
    f32 = jnp.float32
    i32 = jnp.int32
    fn = pl.kernel(
        _edge_kernel_body,
        out_type=(jax.ShapeDtypeStruct((NC, N_PAD, D), f32),
                  jax.ShapeDtypeStruct((NC, DEN_R, D), f32)),
        mesh=mesh,
        scratch_types=(
            [pltpu.VMEM((CH, D), f32)] * 10
            + [pltpu.VMEM((CH,), i32)] * 8
            + [pltpu.VMEM((CH + 16,), i32)] * 4
            + [pltpu.VMEM((CH,), i32)] * 4
            + [pltpu.VMEM((2, 128), i32)] * 2
            + [pltpu.VMEM((256,), f32),
               pltpu.VMEM((256,), i32),
               pltpu.VMEM((ZR, D), f32),
               pltpu.VMEM_SHARED((N_PAD, D), f32),
               pltpu.VMEM_SHARED((DEN_R, D), f32)]
            + [pltpu.SemaphoreType.DMA] * 5
        ),
    )
    return fn(q, k, v, src, dst)


_BR = 1024


def _qkv_body(x_ref, wq_ref, bq_ref, wk_ref, bk_ref, wv_ref, bv_ref,
              q_ref, k_ref, v_ref):
    xb = x_ref[...]
    q_ref[...] = jnp.dot(xb, wq_ref[...], preferred_element_type=jnp.float32) + bq_ref[...]
    k_ref[...] = jnp.dot(xb, wk_ref[...], preferred_element_type=jnp.float32) + bk_ref[...]
    v_ref[...] = jnp.dot(xb, wv_ref[...], preferred_element_type=jnp.float32) + bv_ref[...]


def _qkv(x, Wqt, bq, Wkt, bk, Wvt, bv):
    grid = (N_PAD // _BR,)
    row_spec = pl.BlockSpec((_BR, D), lambda i: (i, 0))
    w_spec = pl.BlockSpec((D, D), lambda i: (0, 0))
    b_spec = pl.BlockSpec((1, D), lambda i: (0, 0))
    return pl.pallas_call(
        _qkv_body,
        grid=grid,
        in_specs=[row_spec, w_spec, b_spec, w_spec, b_spec, w_spec, b_spec],
        out_specs=[row_spec, row_spec, row_spec],
        out_shape=[jax.ShapeDtypeStruct((N_PAD, D), jnp.float32)] * 3,
    )(x, Wqt, bq.reshape(1, D), Wkt, bk.reshape(1, D), Wvt, bv.reshape(1, D))


def _epi_body(agg_ref, den_ref, x_ref, ws_ref, bs_ref, h_ref):
    agg = agg_ref[0] + agg_ref[1]
    denom = den_ref[0] + den_ref[1] + np.float32(1e-16)
    h = agg / denom + jnp.dot(x_ref[...], ws_ref[...],
                              preferred_element_type=jnp.float32) + bs_ref[...]
    h_ref[...] = jnp.maximum(h, 0.0)


def _epilogue(agg, den_col, x, Wst, bs):
    grid = (N_PAD // _BR,)
    return pl.pallas_call(
        _epi_body,
        grid=grid,
        in_specs=[
            pl.BlockSpec((NC, _BR, D), lambda i: (0, i, 0)),
            pl.BlockSpec((NC, _BR, 1), lambda i: (0, i, 0)),
            pl.BlockSpec((_BR, D), lambda i: (i, 0)),
            pl.BlockSpec((D, D), lambda i: (0, 0)),
            pl.BlockSpec((1, D), lambda i: (0, 0)),
        ],
        out_specs=pl.BlockSpec((_BR, D), lambda i: (i, 0)),
        out_shape=jax.ShapeDtypeStruct((N_PAD, D), jnp.float32),
    )(agg, den_col, x, Wst, bs.reshape(1, D))


def _head_body(h_ref, w1_ref, b1_ref, w2_ref, b2_ref, o_ref):
    g = jnp.mean(h_ref[...], axis=0, keepdims=True)
    z = jnp.dot(g, w1_ref[...], preferred_element_type=jnp.float32) + b1_ref[...]
    z = jnp.maximum(z * _LN_C, 0.0)
    o_ref[...] = jnp.dot(z, w2_ref[...], preferred_element_type=jnp.float32) + b2_ref[...]


def _head(h, Wc1t, bc1, Wc2t, bc2):
    return pl.pallas_call(
        _head_body,
        out_shape=jax.ShapeDtypeStruct((1, OUT), jnp.float32),
    )(h, Wc1t, bc1.reshape(1, D // 2), Wc2t, bc2.reshape(1, OUT))


def kernel(x, edge_index, Wq0, bq0, Wk0, bk0, Wv0, bv0, Ws0, bs0,
           Wq1, bq1, Wk1, bk1, Wv1, bv1, Ws1, bs1, Wc1, bc1, Wc2, bc2):
    pad = jnp.full((E_PAD - E,), N, jnp.int32)
    src = jnp.concatenate([edge_index[0], pad]).reshape(NW, IDXR, 128)
    dst = jnp.concatenate([edge_index[1], pad]).reshape(NW, IDXR, 128)
    xp = jnp.pad(x, ((0, N_PAD - N), (0, 0)))

    def layer(h, Wq, bq, Wk, bk, Wv, bv, Ws, bs):
        q, k, v = _qkv(h, Wq.T, bq, Wk.T, bk, Wv.T, bv)
        agg, den = _edge_partials(q, k, v, src, dst)
        den_col = den.reshape(NC, N_PAD, 1)
        return _epilogue(agg, den_col, h, Ws.T, bs)

    h1 = layer(xp, Wq0, bq0, Wk0, bk0, Wv0, bv0, Ws0, bs0)
    h2 = layer(h1, Wq1, bq1, Wk1, bk1, Wv1, bv1, Ws1, bs1)
    return _head(h2[:N], Wc1.T, bc1, Wc2.T, bc2)


# speedup vs baseline: 1.7912x; 1.0425x over previous
import jax
import jax.numpy as jnp
import numpy as np
from jax import lax
from jax.experimental import pallas as pl
from jax.experimental.pallas import tpu as pltpu
from jax.experimental.pallas import tpu_sc as plsc

N = 10000
E = 320000
D = 128
OUT = 10

NC = 2
NS = 16
NW = NC * NS
CH = 32
IDXR = 79
EW = IDXR * 128
E_PAD = NW * EW
N_PAD = 10240
ROWS_PER_TILE = N_PAD // NS
ZR = 8
DEN_R = N_PAD // D

_SCALE = np.float32(1.0 / np.sqrt(np.float32(D)))
_LN_C = np.float32(1.0 / np.sqrt(1.0 + 1e-05))


def _edge_kernel_body(q_hbm, k_hbm, v_hbm, src_hbm, dst_hbm,
                      agg_hbm, den_hbm, *refs):
    (qr0, qr1, kr0, kr1, vr0, vr1, ob0, ob1, oh0, oh1,
     sc0, sc1, sc2, sc3, dc0, dc1, dc2, dc3, dx0, dx1, dx2, dx3,
     dh0, dh1, dh2, dh3, rbs_v, rbd_v,
     red_f, red_i, zbuf, acc, den,
     sem_g0, sem_g1, sem_s0, sem_s1, sem_r) = refs
    qr = (qr0, qr1)
    kr = (kr0, kr1)
    vr = (vr0, vr1)
    ob = (ob0, ob1)
    oh = (oh0, oh1)
    scs = (sc0, sc1, sc2, sc3)
    dcs = (dc0, dc1, dc2, dc3)
    dxs = (dx0, dx1, dx2, dx3)
    dhs = (dh0, dh1, dh2, dh3)
    sem_g = (sem_g0, sem_g1)
    sem_s = (sem_s0, sem_s1)

    cid = lax.axis_index("c")
    sid = lax.axis_index("s")
    wid = sid * NC + cid

    zero16 = jnp.zeros((16,), jnp.float32)
    iota16 = lax.iota(jnp.int32, 16)

    def _zfill(i, _):
        for j in range(D // 16):
            zbuf[i, pl.ds(16 * j, 16)] = zero16
        return 0
    lax.fori_loop(0, ZR, _zfill, 0, unroll=4)

    for t in range(ROWS_PER_TILE // ZR):
        pltpu.sync_copy(zbuf, acc.at[pl.ds(sid * ROWS_PER_TILE + t * ZR, ZR)])

    @pl.when(sid < DEN_R // 8)
    def _():
        pltpu.sync_copy(zbuf, den.at[pl.ds(sid * 8, 8)])
    plsc.subcore_barrier()

    def _prep(row, k4p):
        rsel = row % 2
        base = k4p * CH
        for g in range(CH // 16):
            dv = rbd_v[rsel, pl.ds(base + 16 * g, 16)]
            sv = rbs_v[rsel, pl.ds(base + 16 * g, 16)]
            scs[k4p][pl.ds(16 * g, 16)] = sv
            dcs[k4p][pl.ds(16 * g, 16)] = dv
            dxs[k4p][pl.ds(16 * g, 16)] = dv
            dhs[k4p][pl.ds(16 * g, 16)] = lax.shift_right_logical(dv, 7)

    def _issue_gathers(k4p, s):
        cq = pltpu.async_copy(q_hbm.at[dcs[k4p]], qr[s], sem_g[s])
        ck = pltpu.async_copy(k_hbm.at[scs[k4p]], kr[s], sem_g[s])
        cv = pltpu.async_copy(v_hbm.at[scs[k4p]], vr[s], sem_g[s])
        return cq, ck, cv

    def _wait_gathers(k4p, s):
        pltpu.make_async_copy(q_hbm.at[dcs[k4p]], qr[s], sem_g[s]).wait()
        pltpu.make_async_copy(k_hbm.at[scs[k4p]], kr[s], sem_g[s]).wait()
        pltpu.make_async_copy(v_hbm.at[scs[k4p]], vr[s], sem_g[s]).wait()

    def _issue_scatters(k4p, s):
        pltpu.async_copy(ob[s], acc.at[dcs[k4p]], sem_s[s], add=True)
        pltpu.async_copy(oh[s], den.at[dhs[k4p]], sem_s[s], add=True)

    def _wait_scatters(k4p, s):
        pltpu.make_async_copy(ob[s], acc.at[dcs[k4p]], sem_s[s]).wait()
        pltpu.make_async_copy(oh[s], den.at[dhs[k4p]], sem_s[s]).wait()

    def _compute(k4p, s):
        U = 4

        def _edge4(eg, _):
            accs = []
            for u in range(U):
                e = eg * U + u
                a = qr[s][e, pl.ds(0, 16)] * kr[s][e, pl.ds(0, 16)]
                for j in range(1, D // 16):
                    a = a + (qr[s][e, pl.ds(16 * j, 16)]
                             * kr[s][e, pl.ds(16 * j, 16)])
                accs.append(a)
            for sh in (8, 4, 2, 1):
                for u in range(U):
                    red_f[pl.ds(u * 32, 16)] = accs[u]
                    red_f[pl.ds(u * 32 + 16, 16)] = accs[u]
                for u in range(U):
                    accs[u] = accs[u] + red_f[pl.ds(u * 32 + sh, 16)]
            exs = [jnp.exp(a * _SCALE) for a in accs]
            dsp = [dxs[k4p][pl.ds(eg * U + u, 16)] for u in range(U)]
            for sh in (1, 2, 4, 8):
                for u in range(U):
                    red_i[pl.ds(u * 32 + sh, 16)] = dsp[u]
                for u in range(U):
                    dsp[u] = jnp.where((iota16 >= sh) & (iota16 < 2 * sh),
                                       red_i[pl.ds(u * 32, 16)], dsp[u])
            for u in range(U):
                e = eg * U + u
                rlow = dsp[u] & (D - 1)
                ex = exs[u]
                for j in range(D // 16):
                    ob[s][e, pl.ds(16 * j, 16)] = ex * vr[s][e, pl.ds(16 * j, 16)]
                    oh[s][e, pl.ds(16 * j, 16)] = jnp.where(
                        iota16 + 16 * j == rlow, ex, zero16)
            return 0

        lax.fori_loop(0, CH // U, _edge4, 0)

    pltpu.sync_copy(src_hbm.at[wid, 0], rbs_v.at[0])
    pltpu.sync_copy(dst_hbm.at[wid, 0], rbd_v.at[0])
    pltpu.async_copy(src_hbm.at[wid, 1], rbs_v.at[1], sem_r)
    pltpu.async_copy(dst_hbm.at[wid, 1], rbd_v.at[1], sem_r)
    _prep(0, 0)
    _issue_gathers(0, 0)

    def _row(row, _):
        for k4 in range(4):
            s = k4 % 2
            if k4 == 3:
                @pl.when(row < IDXR - 1)
                def _():
                    pltpu.make_async_copy(src_hbm.at[wid, 0],
                                          rbs_v.at[0], sem_r).wait()
                    pltpu.make_async_copy(dst_hbm.at[wid, 0],
                                          rbd_v.at[0], sem_r).wait()
                    _prep(row + 1, 0)
                    _issue_gathers(0, 1 - s)

                @pl.when(row < IDXR - 2)
                def _():
                    pltpu.async_copy(src_hbm.at[wid, row + 2],
                                     rbs_v.at[row % 2], sem_r)
                    pltpu.async_copy(dst_hbm.at[wid, row + 2],
                                     rbd_v.at[row % 2], sem_r)
            else:
                _prep(row, k4 + 1)
                _issue_gathers(k4 + 1, 1 - s)
            _wait_gathers(k4, s)
            if k4 >= 2:
                _wait_scatters(k4 - 2, s)
            else:
                @pl.when(row > 0)
                def _():
                    _wait_scatters(k4 + 2, s)
            _compute(k4, s)
            _issue_scatters(k4, s)
        return 0

    lax.fori_loop(0, IDXR, _row, 0)
    _wait_scatters(2, 0)
    _wait_scatters(3, 1)

    plsc.subcore_barrier()
    pltpu.sync_copy(
        acc.at[pl.ds(sid * ROWS_PER_TILE, ROWS_PER_TILE)],
        agg_hbm.at[cid, pl.ds(sid * ROWS_PER_TILE, ROWS_PER_TILE)])

    @pl.when(sid < DEN_R // 8)
    def _():
        pltpu.sync_copy(den.at[pl.ds(sid * 8, 8)],
                        den_hbm.at[cid, pl.ds(sid * 8, 8)])


def _edge_partials(q, k, v, src, dst):
    mesh = plsc.VectorSubcoreMesh(core_axis_name="c", subcore_axis_name="s",
                                  num_cores=NC, num_subcores=NS)
    f32 = jnp.float32
    i32 = jnp.int32
    fn = pl.kernel(
        _edge_kernel_body,
        out_type=(jax.ShapeDtypeStruct((NC, N_PAD, D), f32),
                  jax.ShapeDtypeStruct((NC, DEN_R, D), f32)),
        mesh=mesh,
        scratch_types=(
            [pltpu.VMEM((CH, D), f32)] * 10
            + [pltpu.VMEM((CH,), i32)] * 8
            + [pltpu.VMEM((CH + 16,), i32)] * 4
            + [pltpu.VMEM((CH,), i32)] * 4
            + [pltpu.VMEM((2, 128), i32)] * 2
            + [pltpu.VMEM((128,), f32),
               pltpu.VMEM((128,), i32),
               pltpu.VMEM((ZR, D), f32),
               pltpu.VMEM_SHARED((N_PAD, D), f32),
               pltpu.VMEM_SHARED((DEN_R, D), f32)]
            + [pltpu.SemaphoreType.DMA] * 5
        ),
    )
    return fn(q, k, v, src, dst)


_BR = 1024


def _qkv_body(x_ref, wq_ref, bq_ref, wk_ref, bk_ref, wv_ref, bv_ref,
              q_ref, k_ref, v_ref):
    xb = x_ref[...]
    q_ref[...] = jnp.dot(xb, wq_ref[...], preferred_element_type=jnp.float32) + bq_ref[...]
    k_ref[...] = jnp.dot(xb, wk_ref[...], preferred_element_type=jnp.float32) + bk_ref[...]
    v_ref[...] = jnp.dot(xb, wv_ref[...], preferred_element_type=jnp.float32) + bv_ref[...]


def _qkv(x, Wqt, bq, Wkt, bk, Wvt, bv):
    grid = (N_PAD // _BR,)
    row_spec = pl.BlockSpec((_BR, D), lambda i: (i, 0))
    w_spec = pl.BlockSpec((D, D), lambda i: (0, 0))
    b_spec = pl.BlockSpec((1, D), lambda i: (0, 0))
    return pl.pallas_call(
        _qkv_body,
        grid=grid,
        in_specs=[row_spec, w_spec, b_spec, w_spec, b_spec, w_spec, b_spec],
        out_specs=[row_spec, row_spec, row_spec],
        out_shape=[jax.ShapeDtypeStruct((N_PAD, D), jnp.float32)] * 3,
    )(x, Wqt, bq.reshape(1, D), Wkt, bk.reshape(1, D), Wvt, bv.reshape(1, D))


def _epi_body(agg_ref, den_ref, x_ref, ws_ref, bs_ref, h_ref):
    agg = agg_ref[0] + agg_ref[1]
    denom = den_ref[0] + den_ref[1] + np.float32(1e-16)
    h = agg / denom + jnp.dot(x_ref[...], ws_ref[...],
                              preferred_element_type=jnp.float32) + bs_ref[...]
    h_ref[...] = jnp.maximum(h, 0.0)


def _epilogue(agg, den_col, x, Wst, bs):
    grid = (N_PAD // _BR,)
    return pl.pallas_call(
        _epi_body,
        grid=grid,
        in_specs=[
            pl.BlockSpec((NC, _BR, D), lambda i: (0, i, 0)),
            pl.BlockSpec((NC, _BR, 1), lambda i: (0, i, 0)),
            pl.BlockSpec((_BR, D), lambda i: (i, 0)),
            pl.BlockSpec((D, D), lambda i: (0, 0)),
            pl.BlockSpec((1, D), lambda i: (0, 0)),
        ],
        out_specs=pl.BlockSpec((_BR, D), lambda i: (i, 0)),
        out_shape=jax.ShapeDtypeStruct((N_PAD, D), jnp.float32),
    )(agg, den_col, x, Wst, bs.reshape(1, D))


def _head_body(h_ref, w1_ref, b1_ref, w2_ref, b2_ref, o_ref):
    g = jnp.mean(h_ref[...], axis=0, keepdims=True)
    z = jnp.dot(g, w1_ref[...], preferred_element_type=jnp.float32) + b1_ref[...]
    z = jnp.maximum(z * _LN_C, 0.0)
    o_ref[...] = jnp.dot(z, w2_ref[...], preferred_element_type=jnp.float32) + b2_ref[...]


def _head(h, Wc1t, bc1, Wc2t, bc2):
    return pl.pallas_call(
        _head_body,
        out_shape=jax.ShapeDtypeStruct((1, OUT), jnp.float32),
    )(h, Wc1t, bc1.reshape(1, D // 2), Wc2t, bc2.reshape(1, OUT))


def kernel(x, edge_index, Wq0, bq0, Wk0, bk0, Wv0, bv0, Ws0, bs0,
           Wq1, bq1, Wk1, bk1, Wv1, bv1, Ws1, bs1, Wc1, bc1, Wc2, bc2):
    pad = jnp.full((E_PAD - E,), N, jnp.int32)
    src = jnp.concatenate([edge_index[0], pad]).reshape(NW, IDXR, 128)
    dst = jnp.concatenate([edge_index[1], pad]).reshape(NW, IDXR, 128)
    xp = jnp.pad(x, ((0, N_PAD - N), (0, 0)))

    def layer(h, Wq, bq, Wk, bk, Wv, bv, Ws, bs):
        q, k, v = _qkv(h, Wq.T, bq, Wk.T, bk, Wv.T, bv)
        agg, den = _edge_partials(q, k, v, src, dst)
        den_col = den.reshape(NC, N_PAD, 1)
        return _epilogue(agg, den_col, h, Ws.T, bs)

    h1 = layer(xp, Wq0, bq0, Wk0, bk0, Wv0, bv0, Ws0, bs0)
    h2 = layer(h1, Wq1, bq1, Wk1, bk1, Wv1, bv1, Ws1, bs1)
    return _head(h2[:N], Wc1.T, bc1, Wc2.T, bc2)
